# baseline (speedup 1.0000x reference)
import functools

import jax
import jax.numpy as jnp
from jax import lax
from jax.experimental import pallas as pl
from jax.experimental.pallas import tpu as pltpu
from jax.experimental.pallas import tpu_sc as plsc

VOCAB = 1000000
EMBED = 128
B = 4096
L = 200
OUT = 384

NC = 2
NS = 16
NW = NC * NS
RPW = B // NW
FPW = RPW * L
LANES = 16
KCH = EMBED // LANES

CH = 128
CPG = 25
GFLAT = CH * CPG
GROWS = GFLAT // L
NG = FPW // GFLAT
NBUF = 5

_SEGS = []
for _s in range(CPG):
    _f0, _f1 = CH * _s, CH * _s + CH
    _r0 = _f0 // L
    _bd = (_r0 + 1) * L
    if _bd < _f1:
        _SEGS.append([(0, _bd - _f0, _r0), (_bd - _f0, _f1 - _bd, None)])
    elif _bd == _f1:
        _SEGS.append([(0, CH, _r0)])
    else:
        _SEGS.append([(0, CH, None)])

_mesh = plsc.VectorSubcoreMesh(core_axis_name="c", subcore_axis_name="s")


@functools.partial(
    pl.kernel,
    out_type=jax.ShapeDtypeStruct((B, EMBED), jnp.float32),
    mesh=_mesh,
    scratch_types=[
        pltpu.VMEM((2, GFLAT), jnp.int32),
        pltpu.VMEM((CH, EMBED), jnp.float32),
        pltpu.VMEM((CH, EMBED), jnp.float32),
        pltpu.VMEM((CH, EMBED), jnp.float32),
        pltpu.VMEM((CH, EMBED), jnp.float32),
        pltpu.VMEM((CH, EMBED), jnp.float32),
        pltpu.VMEM((GROWS, EMBED), jnp.float32),
        pltpu.SemaphoreType.DMA,
        pltpu.SemaphoreType.DMA,
        pltpu.SemaphoreType.DMA,
        pltpu.SemaphoreType.DMA,
        pltpu.SemaphoreType.DMA,
        pltpu.SemaphoreType.DMA,
    ],
)
def _bag_kernel(idx_hbm, table_hbm, out_hbm, idxc, buf0, buf1, buf2, buf3,
                buf4, fbuf, sem0, sem1, sem2, sem3, sem4, isem):
    wid = lax.axis_index("s") * NC + lax.axis_index("c")
    fbase = wid * FPW
    rbase = wid * RPW

    bufs = (buf0, buf1, buf2, buf3, buf4)
    sems = (sem0, sem1, sem2, sem3, sem4)

    pltpu.sync_copy(idx_hbm.at[pl.ds(fbase, GFLAT)], idxc.at[0])
    pltpu.async_copy(idx_hbm.at[pl.ds(fbase + GFLAT, GFLAT)], idxc.at[1], isem)

    def start(p, off, b):
        pltpu.async_copy(
            table_hbm.at[idxc.at[p, pl.ds(off, CH)]], bufs[b], sems[b])

    def wait(b):
        pltpu.make_async_copy(
            table_hbm.at[idxc.at[0, pl.ds(0, CH)]], bufs[b], sems[b]).wait()

    def seg_sum(buf, off, ln, acc):
        def body(i, a):
            aa = list(a)
            for u in range(4):
                j = off + 4 * i + u
                for k in range(KCH):
                    aa[k] = aa[k] + buf[j, pl.ds(k * LANES, LANES)]
            return tuple(aa)

        return lax.fori_loop(0, ln // 4, body, acc)

    zeros = tuple(jnp.zeros((LANES,), jnp.float32) for _ in range(KCH))

    for s in range(NBUF):
        start(0, CH * s, s)

    def group(g, carry):
        acc = zeros
        for s in range(CPG):
            b = s % NBUF
            if s == 20:
                @pl.when(g <= NG - 2)
                def _():
                    pltpu.make_async_copy(
                        idx_hbm.at[pl.ds(0, GFLAT)], idxc.at[0], isem).wait()

            wait(b)
            for off, ln, fl in _SEGS[s]:
                acc = seg_sum(bufs[b], off, ln, acc)
                if fl is not None:
                    for k in range(KCH):
                        fbuf[fl, pl.ds(k * LANES, LANES)] = (
                            acc[k] * jnp.float32(1.0 / L))
                    acc = zeros

            if s + NBUF < CPG:
                start(g % 2, CH * (s + NBUF), b)
            else:
                @pl.when(g <= NG - 2)
                def _():
                    start((g + 1) % 2, CH * (s + NBUF - CPG), b)

        pltpu.sync_copy(fbuf, out_hbm.at[pl.ds(rbase + GROWS * g, GROWS)])

        @pl.when(g <= NG - 3)
        def _():
            pltpu.async_copy(
                idx_hbm.at[pl.ds(fbase + GFLAT * (g + 2), GFLAT)],
                idxc.at[g % 2], isem)

        return carry

    lax.fori_loop(0, NG, group, 0)


def _mlp_body(text, url, link, Wu, bu, Wl, bl, W1t, W1u, W1l, bc1, Wc2, bc2, out):
    f32 = jnp.float32
    u = jnp.maximum(jnp.dot(url[...], Wu[...], preferred_element_type=f32) + bu[...], 0.0)
    lv = jnp.maximum(jnp.dot(link[...], Wl[...], preferred_element_type=f32) + bl[...], 0.0)
    h = jnp.dot(text[...], W1t[...], preferred_element_type=f32)
    h = h + jnp.dot(u, W1u[...], preferred_element_type=f32)
    h = h + jnp.dot(lv, W1l[...], preferred_element_type=f32)
    h = jnp.maximum(h + bc1[...], 0.0)
    out[...] = jnp.dot(h, Wc2[...], preferred_element_type=f32) + bc2[...]


_BB = 512


def _mlp(text_vec, url_bits, link_feats, Wu, bu, Wl, bl, W1t, W1u, W1l, bc1, Wc2, bc2):
    n = B // _BB
    row = lambda i: (i, 0)
    rep = lambda i: (0, 0)
    return pl.pallas_call(
        _mlp_body,
        grid=(n,),
        in_specs=[
            pl.BlockSpec((_BB, EMBED), row),
            pl.BlockSpec((_BB, 64), row),
            pl.BlockSpec((_BB, 32), row),
            pl.BlockSpec((64, 64), rep),
            pl.BlockSpec((1, 64), rep),
            pl.BlockSpec((32, 64), rep),
            pl.BlockSpec((1, 64), rep),
            pl.BlockSpec((EMBED, 256), rep),
            pl.BlockSpec((64, 256), rep),
            pl.BlockSpec((64, 256), rep),
            pl.BlockSpec((1, 256), rep),
            pl.BlockSpec((256, OUT), rep),
            pl.BlockSpec((1, OUT), rep),
        ],
        out_specs=pl.BlockSpec((_BB, OUT), row),
        out_shape=jax.ShapeDtypeStruct((B, OUT), jnp.float32),
    )(text_vec, url_bits, link_feats, Wu, bu, Wl, bl, W1t, W1u, W1l, bc1, Wc2, bc2)


def kernel(text_indices, url_bits, link_feats, text_table, Wu, bu, Wl, bl, Wc1, bc1, Wc2, bc2):
    idx = text_indices.astype(jnp.int32).reshape(B * L)
    text_vec = _bag_kernel(idx, text_table)
    W1t = Wc1[:EMBED]
    W1u = Wc1[EMBED:EMBED + 64]
    W1l = Wc1[EMBED + 64:]
    return _mlp(
        text_vec, url_bits, link_feats,
        Wu, bu.reshape(1, 64), Wl, bl.reshape(1, 64),
        W1t, W1u, W1l, bc1.reshape(1, 256), Wc2, bc2.reshape(1, OUT))
